# baseline (speedup 1.0000x reference)
import jax
import jax.numpy as jnp
from jax import lax
from jax.experimental import pallas as pl
from jax.experimental.pallas import tpu as pltpu
from jax.experimental.pallas import tpu_sc as plsc

EMBED_DIM = 32
BATCH = 16384
NUM_CORES = 2
NUM_WORKERS = 32
ROWS_PER_WORKER = BATCH // NUM_WORKERS
LANES = 16
GROUPS = ROWS_PER_WORKER // LANES
CHUNK = 4


def _body(uids_ref, iids_ref, ut_ref, it_ref,
          out_ref, uids_v, iids_v, utile, itile, out_v, sem_u, sem_i):
    wid = lax.axis_index("s") * NUM_CORES + lax.axis_index("c")
    base = wid * ROWS_PER_WORKER

    pltpu.sync_copy(uids_ref.at[pl.ds(base, ROWS_PER_WORKER)], uids_v)
    pltpu.sync_copy(iids_ref.at[pl.ds(base, ROWS_PER_WORKER)], iids_v)

    lane = lax.iota(jnp.int32, LANES)

    def shuffle(x, perm):
        return lax.gather(
            x, perm[:, None],
            lax.GatherDimensionNumbers(
                offset_dims=(), collapsed_slice_dims=(0,),
                start_index_map=(0,)),
            slice_sizes=(1,),
            mode=lax.GatherScatterMode.PROMISE_IN_BOUNDS)

    def group_body(g2, _):
        g = g2 * 2
        vec_u = [uids_v[pl.ds((g + i) * LANES, LANES)] for i in range(2)]
        vec_i = [iids_v[pl.ds((g + i) * LANES, LANES)] for i in range(2)]

        def ids_at(c, k):
            half = c // 4
            j = (c % 4) * CHUNK + k
            return vec_u[half][j], vec_i[half][j]

        def fire_chunk(c):
            p = c % 3
            for k in range(CHUNK):
                cu, ci = ids_at(c, k)
                off_u = pl.multiple_of((cu >> 7) * 128, 128)
                off_i = pl.multiple_of((ci >> 7) * 128, 128)
                for a in range(4):
                    pltpu.async_copy(
                        ut_ref.at[pl.ds(8 * a, 8), pl.ds(off_u, 128)],
                        utile.at[p, k, pl.ds(8 * a, 8)], sem_u)
                    pltpu.async_copy(
                        it_ref.at[pl.ds(8 * a, 8), pl.ds(off_i, 128)],
                        itile.at[p, k, pl.ds(8 * a, 8)], sem_i)

        def wait_chunk(c):
            p = c % 3
            for k in range(CHUNK):
                for a in range(4):
                    pltpu.make_async_copy(
                        ut_ref.at[pl.ds(8 * a, 8), pl.ds(0, 128)],
                        utile.at[p, k, pl.ds(8 * a, 8)], sem_u).wait()
                    pltpu.make_async_copy(
                        it_ref.at[pl.ds(8 * a, 8), pl.ds(0, 128)],
                        itile.at[p, k, pl.ds(8 * a, 8)], sem_i).wait()

        def extract_chunk(c, acc):
            p = c % 3
            for k in range(CHUNK):
                cu, ci = ids_at(c, k)
                pu = jnp.full((LANES,), cu & 15, jnp.int32)
                pi = jnp.full((LANES,), ci & 15, jnp.int32)
                o16u = ((cu >> 4) & 7) * 16
                o16i = ((ci >> 4) & 7) * 16
                s = jnp.zeros((LANES,), jnp.float32)
                for d in range(EMBED_DIM):
                    uv = utile[p, k, d, pl.ds(o16u, LANES)]
                    iv = itile[p, k, d, pl.ds(o16i, LANES)]
                    s = s + shuffle(uv, pu) * iv
                acc = jnp.where(lane == (c % 4) * CHUNK + k,
                                shuffle(s, pi), acc)
            return acc

        accs = [jnp.zeros((LANES,), jnp.float32) for _ in range(2)]
        fire_chunk(0)
        fire_chunk(1)
        for c in range(8):
            if c + 2 < 8:
                fire_chunk(c + 2)
            wait_chunk(c)
            accs[c // 4] = extract_chunk(c, accs[c // 4])

        for i in range(2):
            out_v[pl.ds((g + i) * LANES, LANES)] = (
                5.0 / (1.0 + jnp.exp(-accs[i])))
        return 0

    lax.fori_loop(0, GROUPS // 2, group_body, 0)

    pltpu.sync_copy(out_v, out_ref.at[wid])


def kernel(user_ids, item_ids, user_table, item_table):
    uids = user_ids.astype(jnp.int32)
    iids = item_ids.astype(jnp.int32)
    ut = user_table.T
    it = item_table.T

    mesh = plsc.VectorSubcoreMesh(core_axis_name="c", subcore_axis_name="s")
    f = pl.kernel(
        _body,
        out_type=jax.ShapeDtypeStruct((NUM_WORKERS, ROWS_PER_WORKER),
                                      jnp.float32),
        mesh=mesh,
        scratch_types=[
            pltpu.VMEM((ROWS_PER_WORKER,), jnp.int32),
            pltpu.VMEM((ROWS_PER_WORKER,), jnp.int32),
            pltpu.VMEM((3, CHUNK, EMBED_DIM, 128), jnp.float32),
            pltpu.VMEM((3, CHUNK, EMBED_DIM, 128), jnp.float32),
            pltpu.VMEM((ROWS_PER_WORKER,), jnp.float32),
            pltpu.SemaphoreType.DMA,
            pltpu.SemaphoreType.DMA,
        ],
        compiler_params=pltpu.CompilerParams(use_tc_tiling_on_sc=True),
    )
    return f(uids, iids, ut, it).reshape(BATCH)
